# Initial kernel scaffold; baseline (speedup 1.0000x reference)
import jax
import jax.numpy as jnp
from jax.experimental import pallas as pl


def kernel(x_user, x_item, params, ei_user_rates_item, ei_item_rev_rates_user):
    raise NotImplementedError("write your pallas kernel here")


# speedup vs baseline: 23.4699x; 23.4699x over previous
import functools
import math

import jax
import jax.numpy as jnp
from jax import lax
from jax.experimental import pallas as pl
from jax.experimental.pallas import tpu as pltpu
from jax.experimental.pallas import tpu_sc as plsc

N = 25000
D = 128
C = 64
H = 2
DH = 32
E = 400000

NC = 2
NS = 16
NW = NC * NS
B = 128
NCHUNK = E // B
NP = 25088
ROWS_PER = NP // NS
AW = 40

R = 1000
GRID = N // R


def _pre_body(x_ref, win_ref, lnw_ref, wcat_ref, bcat_ref,
              xs_ref, q_ref, k_ref, v_ref):
    h = jnp.dot(x_ref[...], win_ref[...], preferred_element_type=jnp.float32)
    h = h + lnw_ref[0:1, :]
    m = jnp.mean(h, axis=-1, keepdims=True)
    v = jnp.mean((h - m) * (h - m), axis=-1, keepdims=True)
    hn = (h - m) / jnp.sqrt(v + 1e-5) * lnw_ref[1:2, :] + lnw_ref[2:3, :]
    xs = jnp.maximum(hn, 0.0)
    qkv = jnp.dot(xs, wcat_ref[...], preferred_element_type=jnp.float32)
    qkv = qkv + bcat_ref[...]
    xs_ref[...] = xs
    q_ref[...] = qkv[:, 0:C]
    k_ref[...] = qkv[:, C:2 * C]
    v_ref[...] = qkv[:, 2 * C:3 * C]


_pre_call = pl.pallas_call(
    _pre_body,
    grid=(GRID,),
    in_specs=[
        pl.BlockSpec((R, D), lambda i: (i, 0)),
        pl.BlockSpec((D, C), lambda i: (0, 0)),
        pl.BlockSpec((3, C), lambda i: (0, 0)),
        pl.BlockSpec((C, 3 * C), lambda i: (0, 0)),
        pl.BlockSpec((1, 3 * C), lambda i: (0, 0)),
    ],
    out_specs=[pl.BlockSpec((R, C), lambda i: (i, 0))] * 4,
    out_shape=[jax.ShapeDtypeStruct((N, C), jnp.float32)] * 4,
)


def _post_body(acc_ref, xs_ref, wo_ref, misc_ref, y_ref):
    m0 = acc_ref[0, :, 0:DH]
    m1 = acc_ref[1, :, 0:DH]
    d0 = acc_ref[0, :, DH:DH + 1]
    d1 = acc_ref[1, :, DH:DH + 1]
    msg = jnp.concatenate([m0 / (d0 + 1e-16), m1 / (d1 + 1e-16)], axis=1)
    o = jnp.dot(jax.nn.gelu(msg), wo_ref[...],
                preferred_element_type=jnp.float32) + misc_ref[0:1, :]
    xs = xs_ref[...]
    sig = misc_ref[1:2, :]
    out = sig * o + (1.0 - sig) * xs
    t = out + xs
    m = jnp.mean(t, axis=-1, keepdims=True)
    v = jnp.mean((t - m) * (t - m), axis=-1, keepdims=True)
    y_ref[...] = (t - m) / jnp.sqrt(v + 1e-5) * misc_ref[2:3, :] + misc_ref[3:4, :]


_post_call = pl.pallas_call(
    _post_body,
    grid=(GRID,),
    in_specs=[
        pl.BlockSpec((2, R, AW), lambda i: (0, i, 0)),
        pl.BlockSpec((R, C), lambda i: (i, 0)),
        pl.BlockSpec((C, C), lambda i: (0, 0)),
        pl.BlockSpec((4, C), lambda i: (0, 0)),
    ],
    out_specs=pl.BlockSpec((R, C), lambda i: (i, 0)),
    out_shape=jax.ShapeDtypeStruct((N, C), jnp.float32),
)


def _edge_body(qd_hbm, ks_hbm, vs_hbm, src_hbm, dst_hbm, z_hbm, out_hbm,
               idx_s, idx_d, idx_g, qr, kr, vr, wr, acc, sq, sk, sv):
    cid = lax.axis_index("c")
    sid = lax.axis_index("s")
    wid = sid * NC + cid

    row0 = sid * ROWS_PER
    pltpu.sync_copy(z_hbm, acc.at[pl.ds(row0, ROWS_PER)])

    def zinit16(i, c):
        wr[i, pl.ds(DH - 8, 16)] = jnp.zeros((16,), jnp.float32)
        return c

    lax.fori_loop(0, B, zinit16, 0)

    plsc.subcore_barrier()

    nloc = (NCHUNK - sid + NS - 1) // NS
    off = cid * N
    lanes = lax.iota(jnp.int32, 16)
    offv = jnp.broadcast_to(off, (16,))

    def chunk(n, carry):
        base = (sid + n * NS) * B
        pltpu.sync_copy(src_hbm.at[pl.ds(base, B)], idx_s)
        pltpu.sync_copy(dst_hbm.at[pl.ds(base, B)], idx_d)
        for j in range(B // 16):
            sl = pl.ds(j * 16, 16)
            idx_s[sl] = idx_s[sl] + offv
            idx_g[sl] = idx_d[sl] + offv
        cq = pltpu.async_copy(qd_hbm.at[idx_g], qr, sq)
        ck = pltpu.async_copy(ks_hbm.at[idx_s], kr, sk)
        cv = pltpu.async_copy(vs_hbm.at[idx_s], vr, sv)
        cq.wait()
        ck.wait()
        cv.wait()

        def group(g, c):
            row = g * 16 + lanes
            a = jnp.zeros((16,), jnp.float32)
            for col in range(DH):
                cv_ = jnp.full((16,), col, jnp.int32)
                a = a + (plsc.load_gather(qr, [row, cv_])
                         * plsc.load_gather(kr, [row, cv_]))
            e = jnp.exp(a)
            for col in range(DH):
                cv_ = jnp.full((16,), col, jnp.int32)
                plsc.store_scatter(wr, [row, cv_],
                                   plsc.load_gather(vr, [row, cv_]) * e)
            plsc.store_scatter(wr, [row, jnp.full((16,), DH, jnp.int32)], e)
            return c

        lax.fori_loop(0, B // 16, group, 0)
        pltpu.sync_copy(wr, acc.at[idx_d], add=True)
        return carry

    lax.fori_loop(0, nloc, chunk, 0)
    plsc.subcore_barrier()
    pltpu.sync_copy(acc.at[pl.ds(row0, ROWS_PER)],
                    out_hbm.at[cid, pl.ds(row0, ROWS_PER)])


@functools.cache
def _edge_call_factory():
    return pl.kernel(
        _edge_body,
        out_type=jax.ShapeDtypeStruct((NC, NP, AW), jnp.float32),
        mesh=plsc.VectorSubcoreMesh(core_axis_name="c", subcore_axis_name="s",
                                    num_cores=NC, num_subcores=NS),
        compiler_params=pltpu.CompilerParams(needs_layout_passes=False,
                                             use_tc_tiling_on_sc=False),
        scratch_types=[
            pltpu.VMEM((B,), jnp.int32),
            pltpu.VMEM((B,), jnp.int32),
            pltpu.VMEM((B,), jnp.int32),
            pltpu.VMEM((B, DH), jnp.float32),
            pltpu.VMEM((B, DH), jnp.float32),
            pltpu.VMEM((B, DH), jnp.float32),
            pltpu.VMEM((B, AW), jnp.float32),
            pltpu.VMEM_SHARED((NP, AW), jnp.float32),
            pltpu.SemaphoreType.DMA,
            pltpu.SemaphoreType.DMA,
            pltpu.SemaphoreType.DMA,
        ],
    )


def _blockdiag(a, scale):
    z = jnp.zeros((C, C), jnp.float32)
    z = z.at[0:DH, 0:DH].set(a[0] * scale[0])
    z = z.at[DH:C, DH:C].set(a[1] * scale[1])
    return z


def _fold(p, rel):
    s = rel["p"] / math.sqrt(DH)
    ablk = _blockdiag(rel["a"], s)
    mblk = _blockdiag(rel["m"], jnp.ones((H,), jnp.float32))
    wcat = jnp.concatenate([p["Wq"], p["Wk"] @ ablk, p["Wv"] @ mblk], axis=1)
    bcat = jnp.concatenate([p["bq"], p["bk"] @ ablk, p["bv"] @ mblk])
    lnw = jnp.stack([p["b_in"], p["g_in"], p["b_ln_in"]])
    return wcat, bcat.reshape(1, 3 * C), lnw


def _misc(p):
    sig = jax.nn.sigmoid(p["skip"])
    return jnp.stack([p["bo"], jnp.full((C,), sig), p["g_out"], p["b_out"]])


def _stack_heads(t):
    return jnp.concatenate([t[:, 0:DH], t[:, DH:C]], axis=0)


def kernel(x_user, x_item, params, ei_user_rates_item, ei_item_rev_rates_user):
    pu, pi = params["user"], params["item"]
    ru, ri = params["rel"]["u2i"], params["rel"]["i2u"]

    wcat_u, bcat_u, lnw_u = _fold(pu, ru)
    wcat_i, bcat_i, lnw_i = _fold(pi, ri)

    xs_u, q_u, krel_u, vrel_u = _pre_call(x_user, pu["W_in"], lnw_u, wcat_u, bcat_u)
    xs_i, q_i, krel_i, vrel_i = _pre_call(x_item, pi["W_in"], lnw_i, wcat_i, bcat_i)

    z = jnp.zeros((ROWS_PER, AW), jnp.float32)
    src_ui = ei_user_rates_item[0].astype(jnp.int32)
    dst_ui = ei_user_rates_item[1].astype(jnp.int32)
    src_iu = ei_item_rev_rates_user[0].astype(jnp.int32)
    dst_iu = ei_item_rev_rates_user[1].astype(jnp.int32)

    edge = _edge_call_factory()
    acc_item = edge(_stack_heads(q_i), _stack_heads(krel_u),
                    _stack_heads(vrel_u), src_ui, dst_ui, z)
    acc_user = edge(_stack_heads(q_u), _stack_heads(krel_i),
                    _stack_heads(vrel_i), src_iu, dst_iu, z)

    y_user = _post_call(acc_user, xs_u, pu["Wo"], _misc(pu))
    y_item = _post_call(acc_item, xs_i, pi["Wo"], _misc(pi))
    return y_user, y_item


# speedup vs baseline: 107.5946x; 4.5844x over previous
import functools
import math

import jax
import jax.numpy as jnp
from jax import lax
from jax.experimental import pallas as pl
from jax.experimental.pallas import tpu as pltpu
from jax.experimental.pallas import tpu_sc as plsc

N = 25000
D = 128
C = 64
H = 2
DH = 32
E = 400000

NC = 2
NS = 16
NW = NC * NS
B = 128
NCHUNK = E // B
NP = 25088
ROWS_PER = NP // NS
AW = 48
NLOCP = 2 * ((NCHUNK + 2 * NS - 1) // (2 * NS))
JUNK = NP - 8

R = 1000
GRID = N // R


def _pre_body(x_ref, win_ref, lnw_ref, wcat_ref, bcat_ref,
              xs_ref, q_ref, k_ref, v_ref):
    h = jnp.dot(x_ref[...], win_ref[...], preferred_element_type=jnp.float32)
    h = h + lnw_ref[0:1, :]
    m = jnp.mean(h, axis=-1, keepdims=True)
    v = jnp.mean((h - m) * (h - m), axis=-1, keepdims=True)
    hn = (h - m) / jnp.sqrt(v + 1e-5) * lnw_ref[1:2, :] + lnw_ref[2:3, :]
    xs = jnp.maximum(hn, 0.0)
    qkv = jnp.dot(xs, wcat_ref[...], preferred_element_type=jnp.float32)
    qkv = qkv + bcat_ref[...]
    xs_ref[...] = xs
    q_ref[...] = qkv[:, 0:C]
    k_ref[...] = qkv[:, C:2 * C]
    v_ref[...] = qkv[:, 2 * C:3 * C]


_pre_call = pl.pallas_call(
    _pre_body,
    grid=(GRID,),
    in_specs=[
        pl.BlockSpec((R, D), lambda i: (i, 0)),
        pl.BlockSpec((D, C), lambda i: (0, 0)),
        pl.BlockSpec((3, C), lambda i: (0, 0)),
        pl.BlockSpec((C, 3 * C), lambda i: (0, 0)),
        pl.BlockSpec((1, 3 * C), lambda i: (0, 0)),
    ],
    out_specs=[pl.BlockSpec((R, C), lambda i: (i, 0))] * 4,
    out_shape=[jax.ShapeDtypeStruct((N, C), jnp.float32)] * 4,
)


def _post_body(acc_ref, xs_ref, wo_ref, misc_ref, y_ref):
    m0 = acc_ref[0, :, 0:DH]
    m1 = acc_ref[1, :, 0:DH]
    d0 = acc_ref[0, :, DH:DH + 1]
    d1 = acc_ref[1, :, DH:DH + 1]
    msg = jnp.concatenate([m0 / (d0 + 1e-16), m1 / (d1 + 1e-16)], axis=1)
    o = jnp.dot(jax.nn.gelu(msg), wo_ref[...],
                preferred_element_type=jnp.float32) + misc_ref[0:1, :]
    xs = xs_ref[...]
    sig = misc_ref[1:2, :]
    out = sig * o + (1.0 - sig) * xs
    t = out + xs
    m = jnp.mean(t, axis=-1, keepdims=True)
    v = jnp.mean((t - m) * (t - m), axis=-1, keepdims=True)
    y_ref[...] = (t - m) / jnp.sqrt(v + 1e-5) * misc_ref[2:3, :] + misc_ref[3:4, :]


_post_call = pl.pallas_call(
    _post_body,
    grid=(GRID,),
    in_specs=[
        pl.BlockSpec((2, R, AW), lambda i: (0, i, 0)),
        pl.BlockSpec((R, C), lambda i: (i, 0)),
        pl.BlockSpec((C, C), lambda i: (0, 0)),
        pl.BlockSpec((4, C), lambda i: (0, 0)),
    ],
    out_specs=pl.BlockSpec((R, C), lambda i: (i, 0)),
    out_shape=jax.ShapeDtypeStruct((N, C), jnp.float32),
)


def _edge_body(qd_hbm, ks_hbm, vs_hbm, src_hbm, dst_hbm, z_hbm, out_hbm,
               isr0, isr1, idr0, idr1, ig0, ig1, is0, is1, sc0, sc1,
               qr0, qr1, kr0, kr1, vr0, vr1, wr0, wr1, acc,
               si0, si1, sq0, sq1, sk0, sk1, sv0, sv1, ss0, ss1):
    isr = (isr0, isr1)
    idr = (idr0, idr1)
    ig = (ig0, ig1)
    isx = (is0, is1)
    sc = (sc0, sc1)
    qr = (qr0, qr1)
    kr = (kr0, kr1)
    vr = (vr0, vr1)
    wr = (wr0, wr1)
    si = (si0, si1)
    sq = (sq0, sq1)
    sk = (sk0, sk1)
    sv = (sv0, sv1)
    ss = (ss0, ss1)

    cid = lax.axis_index("c")
    sid = lax.axis_index("s")
    row0 = sid * ROWS_PER
    pltpu.sync_copy(z_hbm, acc.at[pl.ds(row0, ROWS_PER)])
    plsc.subcore_barrier()

    offv = jnp.broadcast_to(cid * N, (16,))
    lanes = lax.iota(jnp.int32, 16)
    zvec = jnp.zeros((16,), jnp.float32)
    junkv = jnp.full((16,), JUNK, jnp.int32)

    def base_of(m):
        return jnp.minimum(sid + m * NS, NCHUNK - 1) * B

    def idx_start(m, b):
        ba = base_of(m)
        pltpu.async_copy(src_hbm.at[pl.ds(ba, B)], isr[b], si[b])
        pltpu.async_copy(dst_hbm.at[pl.ds(ba, B)], idr[b], si[b])

    def idx_wait(b):
        pltpu.make_async_copy(src_hbm.at[pl.ds(0, B)], isr[b], si[b]).wait()
        pltpu.make_async_copy(dst_hbm.at[pl.ds(0, B)], idr[b], si[b]).wait()

    def addoff(b):
        for j in range(B // 16):
            sl = pl.ds(j * 16, 16)
            ig[b][sl] = idr[b][sl] + offv
            isx[b][sl] = isr[b][sl] + offv

    def gather_start(b):
        pltpu.async_copy(qd_hbm.at[ig[b]], qr[b], sq[b])
        pltpu.async_copy(ks_hbm.at[isx[b]], kr[b], sk[b])
        pltpu.async_copy(vs_hbm.at[isx[b]], vr[b], sv[b])

    def gather_wait(b):
        pltpu.make_async_copy(qd_hbm.at[ig[b]], qr[b], sq[b]).wait()
        pltpu.make_async_copy(ks_hbm.at[isx[b]], kr[b], sk[b]).wait()
        pltpu.make_async_copy(vs_hbm.at[isx[b]], vr[b], sv[b]).wait()

    def scatter_wait(b):
        pltpu.make_async_copy(wr[b], acc.at[sc[b]], ss[b]).wait()

    idx_start(0, 0)
    idx_wait(0)
    addoff(0)
    gather_start(0)
    idx_start(1, 1)

    def iteration(n, p):
        q = 1 - p
        idx_wait(q)
        addoff(q)
        gather_start(q)

        @pl.when(n >= 2)
        def _():
            scatter_wait(p)

        gather_wait(p)

        qrp, krp, vrp, wrp = qr[p], kr[p], vr[p], wr[p]

        @plsc.parallel_loop(0, B, unroll=4)
        def _(i):
            h = (qrp[i, pl.ds(0, 16)] * krp[i, pl.ds(0, 16)]
                 + qrp[i, pl.ds(16, 16)] * krp[i, pl.ds(16, 16)])
            e = jnp.exp(jnp.broadcast_to(jnp.sum(h), (16,)))
            wrp[i, pl.ds(0, 16)] = vrp[i, pl.ds(0, 16)] * e
            wrp[i, pl.ds(16, 16)] = vrp[i, pl.ds(16, 16)] * e
            wrp[i, pl.ds(DH, 16)] = jnp.where(lanes < 1, e, zvec)

        goodv = jnp.broadcast_to(sid + n * NS < NCHUNK, (16,))
        for j in range(B // 16):
            sl = pl.ds(j * 16, 16)
            sc[p][sl] = jnp.where(goodv, idr[p][sl], junkv)
        pltpu.async_copy(wr[p], acc.at[sc[p]], ss[p], add=True)
        idx_start(n + 2, p)

    def pair(m, carry):
        iteration(2 * m, 0)
        iteration(2 * m + 1, 1)
        return carry

    lax.fori_loop(0, NLOCP // 2, pair, 0)

    scatter_wait(0)
    scatter_wait(1)
    gather_wait(0)
    idx_wait(1)
    plsc.subcore_barrier()
    pltpu.sync_copy(acc.at[pl.ds(row0, ROWS_PER)],
                    out_hbm.at[cid, pl.ds(row0, ROWS_PER)])


@functools.cache
def _edge_call_factory():
    return pl.kernel(
        _edge_body,
        out_type=jax.ShapeDtypeStruct((NC, NP, AW), jnp.float32),
        mesh=plsc.VectorSubcoreMesh(core_axis_name="c", subcore_axis_name="s",
                                    num_cores=NC, num_subcores=NS),
        compiler_params=pltpu.CompilerParams(needs_layout_passes=False,
                                             use_tc_tiling_on_sc=False),
        scratch_types=(
            [pltpu.VMEM((B,), jnp.int32)] * 10
            + [pltpu.VMEM((B, DH), jnp.float32)] * 6
            + [pltpu.VMEM((B, AW), jnp.float32)] * 2
            + [pltpu.VMEM_SHARED((NP, AW), jnp.float32)]
            + [pltpu.SemaphoreType.DMA] * 10
        ),
    )


def _blockdiag(a, scale):
    z = jnp.zeros((C, C), jnp.float32)
    z = z.at[0:DH, 0:DH].set(a[0] * scale[0])
    z = z.at[DH:C, DH:C].set(a[1] * scale[1])
    return z


def _fold(p, rel):
    s = rel["p"] / math.sqrt(DH)
    ablk = _blockdiag(rel["a"], s)
    mblk = _blockdiag(rel["m"], jnp.ones((H,), jnp.float32))
    wcat = jnp.concatenate([p["Wq"], p["Wk"] @ ablk, p["Wv"] @ mblk], axis=1)
    bcat = jnp.concatenate([p["bq"], p["bk"] @ ablk, p["bv"] @ mblk])
    lnw = jnp.stack([p["b_in"], p["g_in"], p["b_ln_in"]])
    return wcat, bcat.reshape(1, 3 * C), lnw


def _misc(p):
    sig = jax.nn.sigmoid(p["skip"])
    return jnp.stack([p["bo"], jnp.full((C,), sig), p["g_out"], p["b_out"]])


def _stack_heads(t):
    return jnp.concatenate([t[:, 0:DH], t[:, DH:C]], axis=0)


def kernel(x_user, x_item, params, ei_user_rates_item, ei_item_rev_rates_user):
    pu, pi = params["user"], params["item"]
    ru, ri = params["rel"]["u2i"], params["rel"]["i2u"]

    wcat_u, bcat_u, lnw_u = _fold(pu, ru)
    wcat_i, bcat_i, lnw_i = _fold(pi, ri)

    xs_u, q_u, krel_u, vrel_u = _pre_call(x_user, pu["W_in"], lnw_u, wcat_u, bcat_u)
    xs_i, q_i, krel_i, vrel_i = _pre_call(x_item, pi["W_in"], lnw_i, wcat_i, bcat_i)

    z = jnp.zeros((ROWS_PER, AW), jnp.float32)
    src_ui = ei_user_rates_item[0].astype(jnp.int32)
    dst_ui = ei_user_rates_item[1].astype(jnp.int32)
    src_iu = ei_item_rev_rates_user[0].astype(jnp.int32)
    dst_iu = ei_item_rev_rates_user[1].astype(jnp.int32)

    edge = _edge_call_factory()
    acc_item = edge(_stack_heads(q_i), _stack_heads(krel_u),
                    _stack_heads(vrel_u), src_ui, dst_ui, z)
    acc_user = edge(_stack_heads(q_u), _stack_heads(krel_i),
                    _stack_heads(vrel_i), src_iu, dst_iu, z)

    y_user = _post_call(acc_user, xs_u, pu["Wo"], _misc(pu))
    y_item = _post_call(acc_item, xs_i, pi["Wo"], _misc(pi))
    return y_user, y_item
